# Initial kernel scaffold; baseline (speedup 1.0000x reference)
import jax
import jax.numpy as jnp
from jax.experimental import pallas as pl


def kernel(seq_a, edge_index, edge_weight, embedding, W1, b1, W2, b2):
    raise NotImplementedError("write your pallas kernel here")


# speedup vs baseline: 3.6080x; 3.6080x over previous
import functools

import jax
import jax.numpy as jnp
from jax import lax
from jax.experimental import pallas as pl
from jax.experimental.pallas import tpu as pltpu
from jax.experimental.pallas import tpu_sc as plsc

NC = 2
NS = 16
NW = NC * NS
K = 128
LANES = 16


def _mlp_body(x_ref, w1_ref, b1_ref, w2_ref, b2_ref, o_ref):
    x = x_ref[...]
    h = lax.dot_general(x, w1_ref[...], (((1,), (1,)), ((), ())),
                        preferred_element_type=jnp.float32) + b1_ref[...]
    h = jnp.where(h > 0, h, 0.1 * h)
    o_ref[...] = lax.dot_general(h, w2_ref[...], (((1,), (1,)), ((), ())),
                                 preferred_element_type=jnp.float32) + b2_ref[...]


def _mlp(x, w1, b1, w2, b2):
    n, d = x.shape
    blk = 1000
    grid = (n // blk,)
    return pl.pallas_call(
        _mlp_body,
        grid=grid,
        in_specs=[
            pl.BlockSpec((blk, d), lambda i: (i, 0)),
            pl.BlockSpec((d, d), lambda i: (0, 0)),
            pl.BlockSpec((1, d), lambda i: (0, 0)),
            pl.BlockSpec((d, d), lambda i: (0, 0)),
            pl.BlockSpec((1, d), lambda i: (0, 0)),
        ],
        out_specs=pl.BlockSpec((blk, d), lambda i: (i, 0)),
        out_shape=jax.ShapeDtypeStruct((n, d), jnp.float32),
    )(x, w1, b1.reshape(1, d), w2, b2.reshape(1, d))


def _add_body(a_ref, b_ref, o_ref):
    o_ref[...] = a_ref[...] + b_ref[...]


def _combine(p):
    _, n, d = p.shape
    blk = 1000
    return pl.pallas_call(
        _add_body,
        grid=(n // blk,),
        in_specs=[
            pl.BlockSpec((1, blk, d), lambda i: (0, i, 0)),
            pl.BlockSpec((1, blk, d), lambda i: (1, i, 0)),
        ],
        out_specs=pl.BlockSpec((1, blk, d), lambda i: (0, i, 0)),
        out_shape=jax.ShapeDtypeStruct((1, n, d), jnp.float32),
    )(p, p).reshape(n, d)


def _spmm_sc(h, src, dst, w, n_pad):
    ep = src.shape[0]
    d = h.shape[1]
    epw = ep // NW
    nchunk = epw // K
    rows_pw = n_pad // NS
    zr = 128

    mesh = plsc.VectorSubcoreMesh(core_axis_name="c", subcore_axis_name="s",
                                  num_cores=NC, num_subcores=NS)

    @functools.partial(
        pl.kernel,
        out_type=jax.ShapeDtypeStruct((NC, n_pad, d), jnp.float32),
        mesh=mesh,
        scratch_types=[
            pltpu.VMEM((K,), jnp.int32),
            pltpu.VMEM((K,), jnp.int32),
            pltpu.VMEM((K,), jnp.float32),
            pltpu.VMEM((K, d), jnp.float32),
            pltpu.VMEM((zr, d), jnp.float32),
            pltpu.VMEM_SHARED((n_pad, d), jnp.float32),
        ],
    )
    def spmm(h_hbm, src_hbm, dst_hbm, w_hbm, out_hbm,
             src_v, dst_v, w_v, rows_v, zero_v, acc_sh):
        c = lax.axis_index("c")
        s = lax.axis_index("s")
        wid = c * NS + s

        @pl.loop(0, zr)
        def _(i):
            @pl.loop(0, d // LANES)
            def _(j):
                zero_v[i, pl.ds(j * LANES, LANES)] = jnp.zeros((LANES,), jnp.float32)

        @pl.loop(0, rows_pw // zr)
        def _(t):
            pltpu.sync_copy(zero_v, acc_sh.at[pl.ds(s * rows_pw + t * zr, zr)])

        plsc.subcore_barrier()

        base = wid * epw

        @pl.loop(0, nchunk)
        def _(ci):
            off = base + ci * K
            pltpu.sync_copy(src_hbm.at[pl.ds(off, K)], src_v)
            pltpu.sync_copy(dst_hbm.at[pl.ds(off, K)], dst_v)
            pltpu.sync_copy(w_hbm.at[pl.ds(off, K)], w_v)
            pltpu.sync_copy(h_hbm.at[src_v], rows_v)

            @pl.loop(0, K // LANES)
            def _(g):
                wvec = w_v[pl.ds(g * LANES, LANES)]
                for i in range(LANES):
                    wv = jnp.full((LANES,), wvec[i], jnp.float32)
                    e = g * LANES + i
                    for j in range(d // LANES):
                        sl = pl.ds(j * LANES, LANES)
                        rows_v[e, sl] = rows_v[e, sl] * wv

            pltpu.sync_copy(rows_v, acc_sh.at[dst_v], add=True)

        plsc.subcore_barrier()

        pltpu.sync_copy(acc_sh.at[pl.ds(s * rows_pw, rows_pw)],
                        out_hbm.at[c].at[pl.ds(s * rows_pw, rows_pw)])

    return spmm(h, src, dst, w)


def kernel(seq_a, edge_index, edge_weight, embedding, W1, b1, W2, b2):
    n, d = embedding.shape
    e = edge_weight.shape[0]
    ep = ((e + NW * K - 1) // (NW * K)) * (NW * K)
    pad = ep - e
    dst = jnp.concatenate([edge_index[0], jnp.zeros((pad,), jnp.int32)])
    src = jnp.concatenate([edge_index[1], jnp.zeros((pad,), jnp.int32)])
    w = jnp.concatenate([edge_weight, jnp.zeros((pad,), jnp.float32)])

    n_pad = ((n + NS * 8 - 1) // (NS * 8)) * (NS * 8)
    h_a = lax.optimization_barrier(_mlp(embedding, W1, b1, W2, b2))
    p1 = lax.optimization_barrier(_spmm_sc(h_a, src, dst, w, n_pad))
    m1 = lax.optimization_barrier(_combine(p1))
    p2 = lax.optimization_barrier(_spmm_sc(m1, src, dst, w, n_pad))
    h_p = _combine(p2)
    return h_p[:n]
